# baseline (speedup 1.0000x reference)
import functools

import jax
import jax.numpy as jnp
from jax import lax
from jax.experimental import pallas as pl
from jax.experimental.pallas import tpu as pltpu
from jax.experimental.pallas import tpu_sc as plsc

_NC = 2
_NS = 16
_NW = _NC * _NS
_LANES = 16
_IDX_CHUNK = 128


def _make_gather_mean(vocab, embed, batch, ctx):
  assert batch % _NW == 0
  bpw = batch // _NW
  ipw = bpw * ctx
  assert ipw % _IDX_CHUNK == 0
  nchunk = ipw // _IDX_CHUNK
  nvec = embed // _LANES

  mesh = plsc.VectorSubcoreMesh(core_axis_name="c", subcore_axis_name="s")

  @functools.partial(
      pl.kernel,
      mesh=mesh,
      out_type=jax.ShapeDtypeStruct((batch, embed), jnp.float32),
      scratch_types=[
          pltpu.VMEM((nchunk, _IDX_CHUNK), jnp.int32),
          pltpu.VMEM((ipw, embed), jnp.float32),
          pltpu.VMEM((bpw, embed), jnp.float32),
          pltpu.SemaphoreType.DMA,
      ],
      compiler_params=pltpu.CompilerParams(use_tc_tiling_on_sc=False),
  )
  def gather_mean(idx_hbm, table_hbm, avg_hbm, idx_v, rows_v, avg_v, sem):
    wid = lax.axis_index("s") * _NC + lax.axis_index("c")
    pltpu.sync_copy(idx_hbm.at[wid], idx_v)
    copies = []
    for k in range(nchunk):
      copies.append(
          pltpu.async_copy(
              table_hbm.at[idx_v.at[k]],
              rows_v.at[pl.ds(k * _IDX_CHUNK, _IDX_CHUNK)],
              sem,
          ))
    for c in copies:
      c.wait()

    scale = 1.0 / ctx

    def row_body(r, carry):
      for j in range(nvec):
        sl = pl.ds(j * _LANES, _LANES)
        acc = rows_v[r * ctx, sl]
        for t in range(1, ctx):
          acc = acc + rows_v[r * ctx + t, sl]
        avg_v[r, sl] = acc * scale
      return carry

    lax.fori_loop(0, bpw, row_body, 0)
    pltpu.sync_copy(avg_v, avg_hbm.at[pl.ds(wid * bpw, bpw)])

  return gather_mean


def _matmul_body(avg_ref, w_ref, b_ref, out_ref):
  out_ref[...] = lax.dot_general(
      avg_ref[...], w_ref[...],
      (((1,), (1,)), ((), ())),
      preferred_element_type=jnp.float32,
  ) + b_ref[...]


def _project(avg, lin_w, lin_b, tile_n):
  batch, embed = avg.shape
  vocab = lin_w.shape[0]
  grid = pl.cdiv(vocab, tile_n)
  return pl.pallas_call(
      _matmul_body,
      grid=(grid,),
      in_specs=[
          pl.BlockSpec((batch, embed), lambda i: (0, 0)),
          pl.BlockSpec((tile_n, embed), lambda i: (i, 0)),
          pl.BlockSpec((1, tile_n), lambda i: (0, i)),
      ],
      out_specs=pl.BlockSpec((batch, tile_n), lambda i: (0, i)),
      out_shape=jax.ShapeDtypeStruct((batch, vocab), jnp.float32),
  )(avg, lin_w, lin_b.reshape(1, vocab))


def kernel(inputs, emb_table, lin_w, lin_b):
  batch, ctx = inputs.shape
  vocab, embed = emb_table.shape
  ipw = (batch // _NW) * ctx
  idx = inputs.reshape(-1).astype(jnp.int32)
  idx = idx.reshape(_NW, ipw // _IDX_CHUNK, _IDX_CHUNK)
  avg = _make_gather_mean(vocab, embed, batch, ctx)(idx, emb_table)
  return _project(avg, lin_w, lin_b, tile_n=1024)


# speedup vs baseline: 1.0275x; 1.0275x over previous
import functools

import jax
import jax.numpy as jnp
from jax import lax
from jax.experimental import pallas as pl
from jax.experimental.pallas import tpu as pltpu
from jax.experimental.pallas import tpu_sc as plsc

_NC = 2
_NS = 16
_NW = _NC * _NS
_LANES = 16
_IDX_CHUNK = 128


def _make_gather_mean(vocab, embed, batch, ctx):
  assert batch % _NW == 0
  bpw = batch // _NW
  ipw = bpw * ctx
  assert ipw % _IDX_CHUNK == 0
  nchunk = ipw // _IDX_CHUNK
  nvec = embed // _LANES

  mesh = plsc.VectorSubcoreMesh(core_axis_name="c", subcore_axis_name="s")

  @functools.partial(
      pl.kernel,
      mesh=mesh,
      out_type=jax.ShapeDtypeStruct((batch, embed), jnp.float32),
      scratch_types=[
          pltpu.VMEM((nchunk, _IDX_CHUNK), jnp.int32),
          pltpu.VMEM((ipw, embed), jnp.float32),
          pltpu.VMEM((bpw, embed), jnp.float32),
          pltpu.SemaphoreType.DMA,
      ],
      compiler_params=pltpu.CompilerParams(use_tc_tiling_on_sc=False),
  )
  def gather_mean(idx_hbm, table_hbm, avg_hbm, idx_v, rows_v, avg_v, sem):
    wid = lax.axis_index("s") * _NC + lax.axis_index("c")
    pltpu.sync_copy(idx_hbm.at[wid], idx_v)
    copies = []
    for k in range(nchunk):
      copies.append(
          pltpu.async_copy(
              table_hbm.at[idx_v.at[k]],
              rows_v.at[pl.ds(k * _IDX_CHUNK, _IDX_CHUNK)],
              sem,
          ))
    for c in copies:
      c.wait()

    scale = 1.0 / ctx

    def row_body(r, carry):
      for j in range(nvec):
        sl = pl.ds(j * _LANES, _LANES)
        acc = rows_v[r * ctx, sl]
        for t in range(1, ctx):
          acc = acc + rows_v[r * ctx + t, sl]
        avg_v[r, sl] = acc * scale
      return carry

    lax.fori_loop(0, bpw, row_body, 0)
    pltpu.sync_copy(avg_v, avg_hbm.at[pl.ds(wid * bpw, bpw)])

  return gather_mean


_NBUF = 2
_CHUNK = 512


def _make_project(batch, embed, vocab, tile_n):
  nt = pl.cdiv(vocab, tile_n)
  nchunk = tile_n // _CHUNK
  tail_cols = vocab - (nt - 1) * tile_n
  rem = tail_cols % 128
  tail_chunks = []
  off = 0
  for w in (_CHUNK, 128):
    while (tail_cols - rem) - off >= w:
      tail_chunks.append((off, w))
      off += w
  assert off + rem == tail_cols

  def body(avg_ref, w_ref, b_ref, out_hbm, bufs, tail_buf, sems):
    i = pl.program_id(0)
    slot = lax.rem(i, _NBUF)

    def full_copies(j, sl):
      base = j * tile_n
      return [
          pltpu.make_async_copy(
              bufs.at[sl, :, pl.ds(c * _CHUNK, _CHUNK)],
              out_hbm.at[:, pl.ds(base + c * _CHUNK, _CHUNK)],
              sems.at[sl, c],
          ) for c in range(nchunk)
      ]

    def tail_copies(sl):
      base = (nt - 1) * tile_n
      copies = [
          pltpu.make_async_copy(
              bufs.at[sl, :, pl.ds(off, w)],
              out_hbm.at[:, pl.ds(base + off, w)],
              sems.at[sl, c],
          ) for c, (off, w) in enumerate(tail_chunks)
      ]
      if rem:
        copies.append(
            pltpu.make_async_copy(
                tail_buf,
                out_hbm.at[:, pl.ds(base + tail_cols - rem, rem)],
                sems.at[sl, len(tail_chunks)],
            ))
      return copies

    @pl.when(i >= _NBUF)
    def _drain_prev():
      for c in full_copies(i - _NBUF, slot):
        c.wait()

    bufs[slot] = lax.dot_general(
        avg_ref[...], w_ref[...],
        (((1,), (1,)), ((), ())),
        preferred_element_type=jnp.float32,
    ) + b_ref[...]

    @pl.when(i < nt - 1)
    def _fire_full():
      for c in full_copies(i, slot):
        c.start()

    @pl.when(i == nt - 1)
    def _last_step():
      if rem:
        tail_buf[...] = bufs[slot, :, pl.ds(tail_cols - rem, rem)]
      for c in tail_copies(slot):
        c.start()
      prev = lax.rem(i - 1, _NBUF)
      for c in full_copies(i - 1, prev):
        c.wait()
      for c in tail_copies(slot):
        c.wait()

  return pl.pallas_call(
      body,
      grid=(nt,),
      in_specs=[
          pl.BlockSpec((batch, embed), lambda i: (0, 0)),
          pl.BlockSpec((tile_n, embed), lambda i: (i, 0)),
          pl.BlockSpec((1, tile_n), lambda i: (0, i)),
      ],
      out_specs=pl.BlockSpec(memory_space=pl.ANY),
      out_shape=jax.ShapeDtypeStruct((batch, vocab), jnp.float32),
      scratch_shapes=[
          pltpu.VMEM((_NBUF, batch, tile_n), jnp.float32),
          pltpu.VMEM((batch, max(rem, 1)), jnp.float32),
          pltpu.SemaphoreType.DMA((_NBUF, max(nchunk, len(tail_chunks) + 1))),
      ],
  )


def _project(avg, lin_w, lin_b, tile_n):
  batch, embed = avg.shape
  vocab = lin_w.shape[0]
  return _make_project(batch, embed, vocab, tile_n)(
      avg, lin_w, lin_b.reshape(1, vocab))


def kernel(inputs, emb_table, lin_w, lin_b):
  batch, ctx = inputs.shape
  vocab, embed = emb_table.shape
  ipw = (batch // _NW) * ctx
  idx = inputs.reshape(-1).astype(jnp.int32)
  idx = idx.reshape(_NW, ipw // _IDX_CHUNK, _IDX_CHUNK)
  avg = _make_gather_mean(vocab, embed, batch, ctx)(idx, emb_table)
  return _project(avg, lin_w, lin_b, tile_n=1024)


# speedup vs baseline: 2.2379x; 2.1779x over previous
import functools

import jax
import jax.numpy as jnp
from jax import lax
from jax.experimental import pallas as pl
from jax.experimental.pallas import tpu as pltpu
from jax.experimental.pallas import tpu_sc as plsc

_NC = 2
_NS = 16
_NW = _NC * _NS
_LANES = 16
_IDX_CHUNK = 128


def _make_gather_mean(vocab, embed, batch, ctx):
  assert batch % _NW == 0
  bpw = batch // _NW
  ipw = bpw * ctx
  assert ipw % _IDX_CHUNK == 0
  nchunk = ipw // _IDX_CHUNK
  nvec = embed // _LANES

  mesh = plsc.VectorSubcoreMesh(core_axis_name="c", subcore_axis_name="s")

  @functools.partial(
      pl.kernel,
      mesh=mesh,
      out_type=jax.ShapeDtypeStruct((batch, embed), jnp.float32),
      scratch_types=[
          pltpu.VMEM((nchunk, _IDX_CHUNK), jnp.int32),
          pltpu.VMEM((ipw, embed), jnp.float32),
          pltpu.VMEM((bpw, embed), jnp.float32),
          pltpu.SemaphoreType.DMA,
      ],
      compiler_params=pltpu.CompilerParams(use_tc_tiling_on_sc=False),
  )
  def gather_mean(idx_hbm, table_hbm, avg_hbm, idx_v, rows_v, avg_v, sem):
    wid = lax.axis_index("s") * _NC + lax.axis_index("c")
    pltpu.sync_copy(idx_hbm.at[wid], idx_v)
    copies = []
    for k in range(nchunk):
      copies.append(
          pltpu.async_copy(
              table_hbm.at[idx_v.at[k]],
              rows_v.at[pl.ds(k * _IDX_CHUNK, _IDX_CHUNK)],
              sem,
          ))
    for c in copies:
      c.wait()

    scale = 1.0 / ctx

    def row_body(r, carry):
      for j in range(nvec):
        sl = pl.ds(j * _LANES, _LANES)
        acc = rows_v[r * ctx, sl]
        for t in range(1, ctx):
          acc = acc + rows_v[r * ctx + t, sl]
        avg_v[r, sl] = acc * scale
      return carry

    lax.fori_loop(0, bpw, row_body, 0)
    pltpu.sync_copy(avg_v, avg_hbm.at[pl.ds(wid * bpw, bpw)])

  return gather_mean


_NBUF = 2
_CHUNK = 512


def _make_project(batch, embed, vocab, tile_n):
  nt = pl.cdiv(vocab, tile_n)
  nchunk = tile_n // _CHUNK
  tail_rows = vocab - (nt - 1) * tile_n
  tail_chunks = []
  off = 0
  while tail_rows - off >= _CHUNK:
    tail_chunks.append((off, _CHUNK))
    off += _CHUNK
  if tail_rows - off:
    assert (tail_rows - off) % 8 == 0
    tail_chunks.append((off, tail_rows - off))

  def body(avg_ref, wt_ref, b_ref, out_hbm, bufs, sems):
    i = pl.program_id(0)
    slot = lax.rem(i, _NBUF)

    def full_copies(j, sl):
      base = j * tile_n
      return [
          pltpu.make_async_copy(
              bufs.at[sl, pl.ds(c * _CHUNK, _CHUNK), :],
              out_hbm.at[pl.ds(base + c * _CHUNK, _CHUNK), :],
              sems.at[sl, c],
          ) for c in range(nchunk)
      ]

    def tail_copies(sl):
      base = (nt - 1) * tile_n
      return [
          pltpu.make_async_copy(
              bufs.at[sl, pl.ds(off, w), :],
              out_hbm.at[pl.ds(base + off, w), :],
              sems.at[sl, c],
          ) for c, (off, w) in enumerate(tail_chunks)
      ]

    @pl.when(i >= _NBUF)
    def _drain_prev():
      for c in full_copies(i - _NBUF, slot):
        c.wait()

    bufs[slot] = lax.dot_general(
        wt_ref[...], avg_ref[...],
        (((0,), (1,)), ((), ())),
        preferred_element_type=jnp.float32,
    ) + b_ref[...]

    @pl.when(i < nt - 1)
    def _fire_full():
      for c in full_copies(i, slot):
        c.start()

    @pl.when(i == nt - 1)
    def _last_step():
      for c in tail_copies(slot):
        c.start()
      prev = lax.rem(i - 1, _NBUF)
      for c in full_copies(i - 1, prev):
        c.wait()
      for c in tail_copies(slot):
        c.wait()

  return pl.pallas_call(
      body,
      grid=(nt,),
      in_specs=[
          pl.BlockSpec((batch, embed), lambda i: (0, 0)),
          pl.BlockSpec((embed, tile_n), lambda i: (0, i)),
          pl.BlockSpec((tile_n, 1), lambda i: (i, 0)),
      ],
      out_specs=pl.BlockSpec(memory_space=pl.ANY),
      out_shape=jax.ShapeDtypeStruct((vocab, batch), jnp.float32),
      scratch_shapes=[
          pltpu.VMEM((_NBUF, tile_n, batch), jnp.float32),
          pltpu.SemaphoreType.DMA((_NBUF, max(nchunk, len(tail_chunks)))),
      ],
  )


def _project(avg, lin_w, lin_b, tile_n):
  batch, embed = avg.shape
  vocab = lin_w.shape[0]
  out_t = _make_project(batch, embed, vocab, tile_n)(
      avg, lin_w.T, lin_b.reshape(vocab, 1))
  return out_t.T


def kernel(inputs, emb_table, lin_w, lin_b):
  batch, ctx = inputs.shape
  vocab, embed = emb_table.shape
  ipw = (batch // _NW) * ctx
  idx = inputs.reshape(-1).astype(jnp.int32)
  idx = idx.reshape(_NW, ipw // _IDX_CHUNK, _IDX_CHUNK)
  avg = _make_gather_mean(vocab, embed, batch, ctx)(idx, emb_table)
  return _project(avg, lin_w, lin_b, tile_n=1024)


# speedup vs baseline: 2.7568x; 1.2319x over previous
import functools

import jax
import jax.numpy as jnp
from jax import lax
from jax.experimental import pallas as pl
from jax.experimental.pallas import tpu as pltpu
from jax.experimental.pallas import tpu_sc as plsc

_NC = 2
_NS = 16
_NW = _NC * _NS
_LANES = 16
_IDX_CHUNK = 128


def _make_gather_mean(vocab, embed, batch, ctx):
  assert batch % _NW == 0
  bpw = batch // _NW
  ipw = bpw * ctx
  assert ipw % _IDX_CHUNK == 0
  nchunk = ipw // _IDX_CHUNK
  nvec = embed // _LANES

  mesh = plsc.VectorSubcoreMesh(core_axis_name="c", subcore_axis_name="s")

  @functools.partial(
      pl.kernel,
      mesh=mesh,
      out_type=jax.ShapeDtypeStruct((batch, embed), jnp.float32),
      scratch_types=[
          pltpu.VMEM((nchunk, _IDX_CHUNK), jnp.int32),
          pltpu.VMEM((ipw, embed), jnp.float32),
          pltpu.VMEM((bpw, embed), jnp.float32),
          pltpu.SemaphoreType.DMA,
      ],
      compiler_params=pltpu.CompilerParams(use_tc_tiling_on_sc=False),
  )
  def gather_mean(idx_hbm, table_hbm, avg_hbm, idx_v, rows_v, avg_v, sem):
    wid = lax.axis_index("s") * _NC + lax.axis_index("c")
    pltpu.sync_copy(idx_hbm.at[wid], idx_v)
    copies = []
    for k in range(nchunk):
      copies.append(
          pltpu.async_copy(
              table_hbm.at[idx_v.at[k]],
              rows_v.at[pl.ds(k * _IDX_CHUNK, _IDX_CHUNK)],
              sem,
          ))
    for c in copies:
      c.wait()

    scale = 1.0 / ctx

    def row_body(r, carry):
      for j in range(nvec):
        sl = pl.ds(j * _LANES, _LANES)
        acc = rows_v[r * ctx, sl]
        for t in range(1, ctx):
          acc = acc + rows_v[r * ctx + t, sl]
        avg_v[r, sl] = acc * scale
      return carry

    lax.fori_loop(0, bpw, row_body, 0)
    pltpu.sync_copy(avg_v, avg_hbm.at[pl.ds(wid * bpw, bpw)])

  return gather_mean


_NBUF = 2
_CHUNK = 512


def _make_project(batch, embed, vocab, tile_n):
  nt = pl.cdiv(vocab, tile_n)
  nchunk = tile_n // _CHUNK
  tail_rows = vocab - (nt - 1) * tile_n
  tail_chunks = []
  off = 0
  while tail_rows - off >= _CHUNK:
    tail_chunks.append((off, _CHUNK))
    off += _CHUNK
  if tail_rows - off:
    assert (tail_rows - off) % 8 == 0
    tail_chunks.append((off, tail_rows - off))

  def body(avg_ref, wt_ref, b_ref, out_hbm, bufs, sems):
    i = pl.program_id(0)
    slot = lax.rem(i, _NBUF)

    def full_copies(j, sl):
      base = j * tile_n
      return [
          pltpu.make_async_copy(
              bufs.at[sl, pl.ds(c * _CHUNK, _CHUNK), :],
              out_hbm.at[pl.ds(base + c * _CHUNK, _CHUNK), :],
              sems.at[sl, c],
          ) for c in range(nchunk)
      ]

    def tail_copies(sl):
      base = (nt - 1) * tile_n
      return [
          pltpu.make_async_copy(
              bufs.at[sl, pl.ds(off, w), :],
              out_hbm.at[pl.ds(base + off, w), :],
              sems.at[sl, c],
          ) for c, (off, w) in enumerate(tail_chunks)
      ]

    @pl.when(i >= _NBUF)
    def _drain_prev():
      for c in full_copies(i - _NBUF, slot):
        c.wait()

    bufs[slot] = lax.dot_general(
        wt_ref[...], avg_ref[...],
        (((0,), (1,)), ((), ())),
        preferred_element_type=jnp.float32,
    ) + b_ref[...].reshape(tile_n, 1)

    @pl.when(i < nt - 1)
    def _fire_full():
      for c in full_copies(i, slot):
        c.start()

    @pl.when(i == nt - 1)
    def _last_step():
      for c in tail_copies(slot):
        c.start()
      prev = lax.rem(i - 1, _NBUF)
      for c in full_copies(i - 1, prev):
        c.wait()
      for c in tail_copies(slot):
        c.wait()

  return pl.pallas_call(
      body,
      grid=(nt,),
      in_specs=[
          pl.BlockSpec((batch, embed), lambda i: (0, 0)),
          pl.BlockSpec((embed, tile_n), lambda i: (0, i)),
          pl.BlockSpec((tile_n,), lambda i: (i,)),
      ],
      out_specs=pl.BlockSpec(memory_space=pl.ANY),
      out_shape=jax.ShapeDtypeStruct((vocab, batch), jnp.float32),
      scratch_shapes=[
          pltpu.VMEM((_NBUF, tile_n, batch), jnp.float32),
          pltpu.SemaphoreType.DMA((_NBUF, max(nchunk, len(tail_chunks)))),
      ],
  )


def _project(avg, lin_w, lin_b, tile_n):
  batch, embed = avg.shape
  vocab = lin_w.shape[0]
  out_t = _make_project(batch, embed, vocab, tile_n)(avg, lin_w.T, lin_b)
  return out_t.T


def kernel(inputs, emb_table, lin_w, lin_b):
  batch, ctx = inputs.shape
  vocab, embed = emb_table.shape
  ipw = (batch // _NW) * ctx
  idx = inputs.reshape(-1).astype(jnp.int32)
  idx = idx.reshape(_NW, ipw // _IDX_CHUNK, _IDX_CHUNK)
  avg = _make_gather_mean(vocab, embed, batch, ctx)(idx, emb_table)
  return _project(avg, lin_w, lin_b, tile_n=1024)


# speedup vs baseline: 2.7781x; 1.0077x over previous
import functools

import jax
import jax.numpy as jnp
from jax import lax
from jax.experimental import pallas as pl
from jax.experimental.pallas import tpu as pltpu
from jax.experimental.pallas import tpu_sc as plsc

_NC = 2
_NS = 16
_NW = _NC * _NS
_LANES = 16
_IDX_CHUNK = 128


def _make_gather_mean(vocab, embed, batch, ctx):
  assert batch % _NW == 0
  bpw = batch // _NW
  ipw = bpw * ctx
  assert ipw % _IDX_CHUNK == 0
  nchunk = ipw // _IDX_CHUNK
  nvec = embed // _LANES

  mesh = plsc.VectorSubcoreMesh(core_axis_name="c", subcore_axis_name="s")

  @functools.partial(
      pl.kernel,
      mesh=mesh,
      out_type=jax.ShapeDtypeStruct((batch, embed), jnp.float32),
      scratch_types=[
          pltpu.VMEM((nchunk, _IDX_CHUNK), jnp.int32),
          pltpu.VMEM((ipw, embed), jnp.float32),
          pltpu.VMEM((bpw, embed), jnp.float32),
          pltpu.SemaphoreType.DMA,
      ],
      compiler_params=pltpu.CompilerParams(use_tc_tiling_on_sc=False),
  )
  def gather_mean(idx_hbm, table_hbm, avg_hbm, idx_v, rows_v, avg_v, sem):
    wid = lax.axis_index("s") * _NC + lax.axis_index("c")
    pltpu.sync_copy(idx_hbm.at[wid], idx_v)
    copies = []
    for k in range(nchunk):
      copies.append(
          pltpu.async_copy(
              table_hbm.at[idx_v.at[k]],
              rows_v.at[pl.ds(k * _IDX_CHUNK, _IDX_CHUNK)],
              sem,
          ))
    for c in copies:
      c.wait()

    scale = 1.0 / ctx

    def row_body(r, carry):
      for j in range(nvec):
        sl = pl.ds(j * _LANES, _LANES)
        acc = rows_v[r * ctx, sl]
        for t in range(1, ctx):
          acc = acc + rows_v[r * ctx + t, sl]
        avg_v[r, sl] = acc * scale
      return carry

    lax.fori_loop(0, bpw, row_body, 0)
    pltpu.sync_copy(avg_v, avg_hbm.at[pl.ds(wid * bpw, bpw)])

  return gather_mean


_NBUF = 2
_CHUNK = 256


def _make_project(batch, embed, vocab, tile_n):
  nt = pl.cdiv(vocab, tile_n)
  nchunk = tile_n // _CHUNK
  tail_rows = vocab - (nt - 1) * tile_n
  tail_chunks = []
  off = 0
  while tail_rows - off >= _CHUNK:
    tail_chunks.append((off, _CHUNK))
    off += _CHUNK
  if tail_rows - off:
    assert (tail_rows - off) % 8 == 0
    tail_chunks.append((off, tail_rows - off))

  def body(avg_ref, wt_ref, b_ref, out_hbm, bufs, sems):
    i = pl.program_id(0)
    slot = lax.rem(i, _NBUF)

    def full_copies(j, sl):
      base = j * tile_n
      return [
          pltpu.make_async_copy(
              bufs.at[sl, pl.ds(c * _CHUNK, _CHUNK), :],
              out_hbm.at[pl.ds(base + c * _CHUNK, _CHUNK), :],
              sems.at[sl, c],
          ) for c in range(nchunk)
      ]

    def tail_copies(sl):
      base = (nt - 1) * tile_n
      return [
          pltpu.make_async_copy(
              bufs.at[sl, pl.ds(off, w), :],
              out_hbm.at[pl.ds(base + off, w), :],
              sems.at[sl, c],
          ) for c, (off, w) in enumerate(tail_chunks)
      ]

    @pl.when(i >= _NBUF)
    def _drain_prev():
      for c in full_copies(i - _NBUF, slot):
        c.wait()

    bufs[slot] = lax.dot_general(
        wt_ref[...], avg_ref[...],
        (((0,), (1,)), ((), ())),
        preferred_element_type=jnp.float32,
    ) + b_ref[...].reshape(tile_n, 1)

    @pl.when(i < nt - 1)
    def _fire_full():
      for c in full_copies(i, slot):
        c.start()

    @pl.when(i == nt - 1)
    def _last_step():
      for c in tail_copies(slot):
        c.start()
      prev = lax.rem(i - 1, _NBUF)
      for c in full_copies(i - 1, prev):
        c.wait()
      for c in tail_copies(slot):
        c.wait()

  return pl.pallas_call(
      body,
      grid=(nt,),
      in_specs=[
          pl.BlockSpec((batch, embed), lambda i: (0, 0)),
          pl.BlockSpec((embed, tile_n), lambda i: (0, i)),
          pl.BlockSpec((tile_n,), lambda i: (i,)),
      ],
      out_specs=pl.BlockSpec(memory_space=pl.ANY),
      out_shape=jax.ShapeDtypeStruct((vocab, batch), jnp.float32),
      scratch_shapes=[
          pltpu.VMEM((_NBUF, tile_n, batch), jnp.float32),
          pltpu.SemaphoreType.DMA((_NBUF, max(nchunk, len(tail_chunks)))),
      ],
  )


def _project(avg, lin_w, lin_b, tile_n):
  batch, embed = avg.shape
  vocab = lin_w.shape[0]
  out_t = _make_project(batch, embed, vocab, tile_n)(avg, lin_w.T, lin_b)
  return out_t.T


def kernel(inputs, emb_table, lin_w, lin_b):
  batch, ctx = inputs.shape
  vocab, embed = emb_table.shape
  ipw = (batch // _NW) * ctx
  idx = inputs.reshape(-1).astype(jnp.int32)
  idx = idx.reshape(_NW, ipw // _IDX_CHUNK, _IDX_CHUNK)
  avg = _make_gather_mean(vocab, embed, batch, ctx)(idx, emb_table)
  return _project(avg, lin_w, lin_b, tile_n=1024)


# speedup vs baseline: 2.8658x; 1.0316x over previous
import functools

import jax
import jax.numpy as jnp
from jax import lax
from jax.experimental import pallas as pl
from jax.experimental.pallas import tpu as pltpu
from jax.experimental.pallas import tpu_sc as plsc

_NC = 2
_NS = 16
_NW = _NC * _NS
_LANES = 16
_IDX_CHUNK = 128


def _make_gather_mean(vocab, embed, batch, ctx):
  assert batch % _NW == 0
  bpw = batch // _NW
  ipw = bpw * ctx
  assert ipw % _IDX_CHUNK == 0
  nchunk = ipw // _IDX_CHUNK
  nvec = embed // _LANES

  mesh = plsc.VectorSubcoreMesh(core_axis_name="c", subcore_axis_name="s")

  @functools.partial(
      pl.kernel,
      mesh=mesh,
      out_type=jax.ShapeDtypeStruct((batch, embed), jnp.float32),
      scratch_types=[
          pltpu.VMEM((nchunk, _IDX_CHUNK), jnp.int32),
          pltpu.VMEM((ipw, embed), jnp.float32),
          pltpu.VMEM((bpw, embed), jnp.float32),
          pltpu.SemaphoreType.DMA,
      ],
      compiler_params=pltpu.CompilerParams(use_tc_tiling_on_sc=False),
  )
  def gather_mean(idx_hbm, table_hbm, avg_hbm, idx_v, rows_v, avg_v, sem):
    wid = lax.axis_index("s") * _NC + lax.axis_index("c")
    pltpu.sync_copy(idx_hbm.at[wid], idx_v)
    copies = []
    for k in range(nchunk):
      copies.append(
          pltpu.async_copy(
              table_hbm.at[idx_v.at[k]],
              rows_v.at[pl.ds(k * _IDX_CHUNK, _IDX_CHUNK)],
              sem,
          ))
    for c in copies:
      c.wait()

    scale = 1.0 / ctx

    def row_body(r, carry):
      for j in range(nvec):
        sl = pl.ds(j * _LANES, _LANES)
        acc = rows_v[r * ctx, sl]
        for t in range(1, ctx):
          acc = acc + rows_v[r * ctx + t, sl]
        avg_v[r, sl] = acc * scale
      return carry

    lax.fori_loop(0, bpw, row_body, 0)
    pltpu.sync_copy(avg_v, avg_hbm.at[pl.ds(wid * bpw, bpw)])

  return gather_mean


_NBUF = 2
_CHUNK = 512


def _make_project(batch, embed, vocab, tile_n):
  nt = pl.cdiv(vocab, tile_n)
  nchunk = tile_n // _CHUNK
  tail_rows = vocab - (nt - 1) * tile_n
  tail_chunks = []
  off = 0
  while tail_rows - off >= _CHUNK:
    tail_chunks.append((off, _CHUNK))
    off += _CHUNK
  if tail_rows - off:
    assert (tail_rows - off) % 8 == 0
    tail_chunks.append((off, tail_rows - off))

  def body(avg_ref, wt_ref, b_ref, out_hbm, bufs, sems):
    i = pl.program_id(0)
    slot = lax.rem(i, _NBUF)

    def full_copies(j, sl):
      base = j * tile_n
      return [
          pltpu.make_async_copy(
              bufs.at[sl, pl.ds(c * _CHUNK, _CHUNK), :],
              out_hbm.at[pl.ds(base + c * _CHUNK, _CHUNK), :],
              sems.at[sl, c],
          ) for c in range(nchunk)
      ]

    def tail_copies(sl):
      base = (nt - 1) * tile_n
      return [
          pltpu.make_async_copy(
              bufs.at[sl, pl.ds(off, w), :],
              out_hbm.at[pl.ds(base + off, w), :],
              sems.at[sl, c],
          ) for c, (off, w) in enumerate(tail_chunks)
      ]

    @pl.when(i >= _NBUF)
    def _drain_prev():
      for c in full_copies(i - _NBUF, slot):
        c.wait()

    bufs[slot] = lax.dot_general(
        wt_ref[...], avg_ref[...],
        (((0,), (1,)), ((), ())),
        preferred_element_type=jnp.float32,
    ) + b_ref[...].reshape(tile_n, 1)

    @pl.when(i < nt - 1)
    def _fire_full():
      for c in full_copies(i, slot):
        c.start()

    @pl.when(i == nt - 1)
    def _last_step():
      for c in tail_copies(slot):
        c.start()
      prev = lax.rem(i - 1, _NBUF)
      for c in full_copies(i - 1, prev):
        c.wait()
      for c in tail_copies(slot):
        c.wait()

  return pl.pallas_call(
      body,
      grid=(nt,),
      in_specs=[
          pl.BlockSpec((batch, embed), lambda i: (0, 0)),
          pl.BlockSpec((embed, tile_n), lambda i: (0, i)),
          pl.BlockSpec((tile_n,), lambda i: (i,)),
      ],
      out_specs=pl.BlockSpec(memory_space=pl.ANY),
      out_shape=jax.ShapeDtypeStruct((vocab, batch), jnp.float32),
      scratch_shapes=[
          pltpu.VMEM((_NBUF, tile_n, batch), jnp.float32),
          pltpu.SemaphoreType.DMA((_NBUF, max(nchunk, len(tail_chunks)))),
      ],
  )


def _project(avg, lin_w, lin_b, tile_n):
  batch, embed = avg.shape
  vocab = lin_w.shape[0]
  out_t = _make_project(batch, embed, vocab, tile_n)(avg, lin_w.T, lin_b)
  return out_t.T


_TILE_N = 4096


def kernel(inputs, emb_table, lin_w, lin_b):
  batch, ctx = inputs.shape
  vocab, embed = emb_table.shape
  ipw = (batch // _NW) * ctx
  idx = inputs.reshape(-1).astype(jnp.int32)
  idx = idx.reshape(_NW, ipw // _IDX_CHUNK, _IDX_CHUNK)
  avg = _make_gather_mean(vocab, embed, batch, ctx)(idx, emb_table)
  return _project(avg, lin_w, lin_b, tile_n=_TILE_N)


# speedup vs baseline: 3.7165x; 1.2968x over previous
import functools

import jax
import jax.numpy as jnp
from jax import lax
from jax.experimental import pallas as pl
from jax.experimental.pallas import tpu as pltpu
from jax.experimental.pallas import tpu_sc as plsc

_NC = 2
_NS = 16
_NW = _NC * _NS
_LANES = 16
_IDX_CHUNK = 128


def _make_gather_mean(vocab, embed, batch, ctx):
  assert batch % _NW == 0
  bpw = batch // _NW
  ipw = bpw * ctx
  assert ipw % _IDX_CHUNK == 0
  nchunk = ipw // _IDX_CHUNK
  nvec = embed // _LANES

  mesh = plsc.VectorSubcoreMesh(core_axis_name="c", subcore_axis_name="s")

  @functools.partial(
      pl.kernel,
      mesh=mesh,
      out_type=jax.ShapeDtypeStruct((batch, embed), jnp.float32),
      scratch_types=[
          pltpu.VMEM((nchunk, _IDX_CHUNK), jnp.int32),
          pltpu.VMEM((ipw, embed), jnp.float32),
          pltpu.VMEM((bpw, embed), jnp.float32),
          pltpu.SemaphoreType.DMA,
      ],
      compiler_params=pltpu.CompilerParams(use_tc_tiling_on_sc=False),
  )
  def gather_mean(idx_hbm, table_hbm, avg_hbm, idx_v, rows_v, avg_v, sem):
    wid = lax.axis_index("s") * _NC + lax.axis_index("c")
    pltpu.sync_copy(idx_hbm.at[wid], idx_v)
    copies = []
    for k in range(nchunk):
      copies.append(
          pltpu.async_copy(
              table_hbm.at[idx_v.at[k]],
              rows_v.at[pl.ds(k * _IDX_CHUNK, _IDX_CHUNK)],
              sem,
          ))
    for c in copies:
      c.wait()

    scale = 1.0 / ctx

    def row_body(r, carry):
      for j in range(nvec):
        sl = pl.ds(j * _LANES, _LANES)
        acc = rows_v[r * ctx, sl]
        for t in range(1, ctx):
          acc = acc + rows_v[r * ctx + t, sl]
        avg_v[r, sl] = acc * scale
      return carry

    lax.fori_loop(0, bpw, row_body, 0)
    pltpu.sync_copy(avg_v, avg_hbm.at[pl.ds(wid * bpw, bpw)])

  return gather_mean


def _make_gather_mean_t(vocab, embed, batch, ctx):
  dpw = embed // _NW
  ngrp = batch // _LANES
  mesh = plsc.VectorSubcoreMesh(core_axis_name="c", subcore_axis_name="s")

  @functools.partial(
      pl.kernel,
      mesh=mesh,
      out_type=jax.ShapeDtypeStruct((embed, batch), jnp.float32),
      scratch_types=[
          pltpu.VMEM((vocab,), jnp.float32),
          pltpu.VMEM((ctx, batch), jnp.int32),
          pltpu.VMEM((dpw, batch), jnp.float32),
          pltpu.SemaphoreType.DMA,
      ],
      compiler_params=pltpu.CompilerParams(needs_layout_passes=False),
  )
  def gather_mean_t(idxT_hbm, et_hbm, avgT_hbm, row_v, idx_v, acc_v, sem):
    wid = lax.axis_index("s") * _NC + lax.axis_index("c")
    pltpu.sync_copy(idxT_hbm, idx_v)
    scale = 1.0 / ctx
    for k in range(dpw):
      d = wid * dpw + k
      pltpu.async_copy(et_hbm.at[d], row_v, sem).wait()

      def grp_body(g, carry, k=k):
        base = g * _LANES
        acc = plsc.load_gather(row_v, [idx_v[0, pl.ds(base, _LANES)]])
        for t in range(1, ctx):
          acc = acc + plsc.load_gather(row_v, [idx_v[t, pl.ds(base, _LANES)]])
        acc_v[k, pl.ds(base, _LANES)] = acc * scale
        return carry

      lax.fori_loop(0, ngrp, grp_body, 0)
    pltpu.sync_copy(acc_v, avgT_hbm.at[pl.ds(wid * dpw, dpw)])

  return gather_mean_t


_NBUF = 2
_CHUNK = 512


def _make_project(batch, embed, vocab, tile_n):
  nt = pl.cdiv(vocab, tile_n)
  nchunk = tile_n // _CHUNK
  tail_rows = vocab - (nt - 1) * tile_n
  tail_chunks = []
  off = 0
  while tail_rows - off >= _CHUNK:
    tail_chunks.append((off, _CHUNK))
    off += _CHUNK
  if tail_rows - off:
    assert (tail_rows - off) % 8 == 0
    tail_chunks.append((off, tail_rows - off))

  def body(avg_ref, wt_ref, b_ref, out_hbm, bufs, sems):
    i = pl.program_id(0)
    slot = lax.rem(i, _NBUF)

    def full_copies(j, sl):
      base = j * tile_n
      return [
          pltpu.make_async_copy(
              bufs.at[sl, pl.ds(c * _CHUNK, _CHUNK), :],
              out_hbm.at[pl.ds(base + c * _CHUNK, _CHUNK), :],
              sems.at[sl, c],
          ) for c in range(nchunk)
      ]

    def tail_copies(sl):
      base = (nt - 1) * tile_n
      return [
          pltpu.make_async_copy(
              bufs.at[sl, pl.ds(off, w), :],
              out_hbm.at[pl.ds(base + off, w), :],
              sems.at[sl, c],
          ) for c, (off, w) in enumerate(tail_chunks)
      ]

    @pl.when(i >= _NBUF)
    def _drain_prev():
      for c in full_copies(i - _NBUF, slot):
        c.wait()

    bufs[slot] = lax.dot_general(
        wt_ref[...], avg_ref[...],
        (((0,), (0,)), ((), ())),
        preferred_element_type=jnp.float32,
    ) + b_ref[...].reshape(tile_n, 1)

    @pl.when(i < nt - 1)
    def _fire_full():
      for c in full_copies(i, slot):
        c.start()

    @pl.when(i == nt - 1)
    def _last_step():
      for c in tail_copies(slot):
        c.start()
      prev = lax.rem(i - 1, _NBUF)
      for c in full_copies(i - 1, prev):
        c.wait()
      for c in tail_copies(slot):
        c.wait()

  return pl.pallas_call(
      body,
      grid=(nt,),
      in_specs=[
          pl.BlockSpec((embed, batch), lambda i: (0, 0)),
          pl.BlockSpec((embed, tile_n), lambda i: (0, i)),
          pl.BlockSpec((tile_n,), lambda i: (i,)),
      ],
      out_specs=pl.BlockSpec(memory_space=pl.ANY),
      out_shape=jax.ShapeDtypeStruct((vocab, batch), jnp.float32),
      scratch_shapes=[
          pltpu.VMEM((_NBUF, tile_n, batch), jnp.float32),
          pltpu.SemaphoreType.DMA((_NBUF, max(nchunk, len(tail_chunks)))),
      ],
  )


def _project(avg_t, lin_w, lin_b, tile_n):
  embed, batch = avg_t.shape
  vocab = lin_w.shape[0]
  out_t = _make_project(batch, embed, vocab, tile_n)(avg_t, lin_w.T, lin_b)
  return out_t.T


_TILE_N = 3072


def kernel(inputs, emb_table, lin_w, lin_b):
  batch, ctx = inputs.shape
  vocab, embed = emb_table.shape
  idx_t = inputs.T.astype(jnp.int32)
  et = emb_table.T
  avg_t = _make_gather_mean_t(vocab, embed, batch, ctx)(idx_t, et)
  return _project(avg_t, lin_w, lin_b, tile_n=_TILE_N)


# speedup vs baseline: 3.7241x; 1.0021x over previous
import functools

import jax
import jax.numpy as jnp
from jax import lax
from jax.experimental import pallas as pl
from jax.experimental.pallas import tpu as pltpu
from jax.experimental.pallas import tpu_sc as plsc

_NC = 2
_NS = 16
_NW = _NC * _NS
_LANES = 16


def _make_gather_mean_t(vocab, embed, batch, ctx):
  dpw = embed // _NW
  ngrp = batch // _LANES
  mesh = plsc.VectorSubcoreMesh(core_axis_name="c", subcore_axis_name="s")

  @functools.partial(
      pl.kernel,
      mesh=mesh,
      out_type=jax.ShapeDtypeStruct((embed, batch), jnp.float32),
      scratch_types=[
          pltpu.VMEM((vocab,), jnp.float32),
          pltpu.VMEM((ctx, batch), jnp.int32),
          pltpu.VMEM((dpw, batch), jnp.float32),
          pltpu.SemaphoreType.DMA,
      ],
      compiler_params=pltpu.CompilerParams(needs_layout_passes=False),
  )
  def gather_mean_t(idxT_hbm, et_hbm, avgT_hbm, row_v, idx_v, acc_v, sem):
    wid = lax.axis_index("s") * _NC + lax.axis_index("c")
    pltpu.sync_copy(idxT_hbm, idx_v)
    scale = 1.0 / ctx
    for k in range(dpw):
      d = wid * dpw + k
      pltpu.async_copy(et_hbm.at[d], row_v, sem).wait()

      def grp_body(g, carry, k=k):
        base = g * _LANES
        acc = plsc.load_gather(row_v, [idx_v[0, pl.ds(base, _LANES)]])
        for t in range(1, ctx):
          acc = acc + plsc.load_gather(row_v, [idx_v[t, pl.ds(base, _LANES)]])
        acc_v[k, pl.ds(base, _LANES)] = acc * scale
        return carry

      lax.fori_loop(0, ngrp, grp_body, 0)
    pltpu.sync_copy(acc_v, avgT_hbm.at[pl.ds(wid * dpw, dpw)])

  return gather_mean_t


_NBUF = 2
_CHUNK = 512


def _make_project(batch, embed, vocab, tile_n):
  nt = pl.cdiv(vocab, tile_n)
  nchunk = tile_n // _CHUNK
  tail_rows = vocab - (nt - 1) * tile_n
  tail_chunks = []
  off = 0
  while tail_rows - off >= _CHUNK:
    tail_chunks.append((off, _CHUNK))
    off += _CHUNK
  if tail_rows - off:
    assert (tail_rows - off) % 8 == 0
    tail_chunks.append((off, tail_rows - off))

  def body(avg_ref, wt_ref, b_ref, out_hbm, bufs, sems):
    i = pl.program_id(0)
    slot = lax.rem(i, _NBUF)

    def full_copies(j, sl):
      base = j * tile_n
      return [
          pltpu.make_async_copy(
              bufs.at[sl, pl.ds(c * _CHUNK, _CHUNK), :],
              out_hbm.at[pl.ds(base + c * _CHUNK, _CHUNK), :],
              sems.at[sl, c],
          ) for c in range(nchunk)
      ]

    def tail_copies(sl):
      base = (nt - 1) * tile_n
      return [
          pltpu.make_async_copy(
              bufs.at[sl, pl.ds(off, w), :],
              out_hbm.at[pl.ds(base + off, w), :],
              sems.at[sl, c],
          ) for c, (off, w) in enumerate(tail_chunks)
      ]

    @pl.when(i >= _NBUF)
    def _drain_prev():
      for c in full_copies(i - _NBUF, slot):
        c.wait()

    bufs[slot] = lax.dot_general(
        wt_ref[...], avg_ref[...],
        (((0,), (0,)), ((), ())),
        preferred_element_type=jnp.float32,
    ) + b_ref[...].reshape(tile_n, 1)

    @pl.when(i < nt - 1)
    def _fire_full():
      for c in full_copies(i, slot):
        c.start()

    @pl.when(i == nt - 1)
    def _last_step():
      for c in tail_copies(slot):
        c.start()
      prev = lax.rem(i - 1, _NBUF)
      for c in full_copies(i - 1, prev):
        c.wait()
      for c in tail_copies(slot):
        c.wait()

  return pl.pallas_call(
      body,
      grid=(nt,),
      in_specs=[
          pl.BlockSpec((embed, batch), lambda i: (0, 0)),
          pl.BlockSpec((embed, tile_n), lambda i: (0, i)),
          pl.BlockSpec((tile_n,), lambda i: (i,)),
      ],
      out_specs=pl.BlockSpec(memory_space=pl.ANY),
      out_shape=jax.ShapeDtypeStruct((vocab, batch), jnp.float32),
      scratch_shapes=[
          pltpu.VMEM((_NBUF, tile_n, batch), jnp.float32),
          pltpu.SemaphoreType.DMA((_NBUF, max(nchunk, len(tail_chunks)))),
      ],
  )


def _project(avg_t, lin_w, lin_b, tile_n):
  embed, batch = avg_t.shape
  vocab = lin_w.shape[0]
  out_t = _make_project(batch, embed, vocab, tile_n)(avg_t, lin_w.T, lin_b)
  return out_t.T


_TILE_N = 3072


def kernel(inputs, emb_table, lin_w, lin_b):
  batch, ctx = inputs.shape
  vocab, embed = emb_table.shape
  idx_t = inputs.T.astype(jnp.int32)
  et = emb_table.T
  avg_t = _make_gather_mean_t(vocab, embed, batch, ctx)(idx_t, et)
  return _project(avg_t, lin_w, lin_b, tile_n=_TILE_N)
